# baseline (speedup 1.0000x reference)
import functools

import jax
import jax.numpy as jnp
from jax import lax
from jax.experimental import pallas as pl
from jax.experimental.pallas import tpu as pltpu
from jax.experimental.pallas import tpu_sc as plsc

B = 16
H = 12
HW = 6
T = 577
P = 576
VOTES = 8
TOPK = 12
CONVPAD = 608
KPAD = 592
BIG = 1 << 20


def _sc_body(x_hbm, idx_out, cnt_out, sbuf, vbuf, vbuf2, pbuf, obuf, kbuf,
             ibuf, shared, sem):
    lanes = lax.iota(jnp.int32, 16)
    zeros16 = jnp.zeros((16,), jnp.int32)
    s_id = lax.axis_index("s")
    c_id = lax.axis_index("c")
    sm = c_id * 8 + (s_id >> 1)
    g = s_id & 1
    neg = jnp.full((16,), -2.0, jnp.float32)
    bigv = jnp.full((16,), BIG, jnp.int32)

    copies = [
        pltpu.async_copy(x_hbm.at[sm, g * HW + h, pl.ds(0, 8), :],
                         sbuf.at[pl.ds(h * 8, 8), :], sem)
        for h in range(HW)
    ]
    for cp in copies:
        cp.wait()

    def head_body(h, _):
        rowv = jnp.full((16,), h * 8, jnp.int32)
        plsc.store_scatter(sbuf, [rowv, zeros16], neg, mask=lanes == 0)

        def scan_chunk(ci, carry):
            vmax, vidx = carry
            cols = ci * 16 + lanes
            v = plsc.load_gather(sbuf, [rowv, cols])
            gt = v > vmax
            return jnp.where(gt, v, vmax), jnp.where(gt, cols, vidx)

        vmax, vidx = lax.fori_loop(
            0, P // 16, scan_chunk,
            (jnp.full((16,), -3.0, jnp.float32), jnp.zeros((16,), jnp.int32)))
        v36 = plsc.load_gather(sbuf, [rowv, jnp.full((16,), P, jnp.int32)])
        gt36 = (v36 > vmax) & (lanes == 0)
        vmax = jnp.where(gt36, v36, vmax)
        vidx = jnp.where(gt36, P, vidx)

        def extract(t, carry):
            vmax, vidx, vreg = carry
            m = jnp.max(vmax)
            p = jnp.min(jnp.where(vmax == m, vidx, bigv)
                        .astype(jnp.uint32)).astype(jnp.int32) + zeros16
            vreg = jnp.where(lanes == t, p, vreg)
            plsc.store_scatter(sbuf, [rowv, p], neg, mask=lanes == 0)
            l = p & 15
            j0 = l + 16 * lanes
            j1 = j0 + 256
            j2 = l + 16 * (lanes + 32)
            ok2 = j2 <= P
            j2c = jnp.minimum(j2, P)
            g0 = plsc.load_gather(sbuf, [rowv, j0])
            g1 = plsc.load_gather(sbuf, [rowv, j1])
            g2 = jnp.where(ok2, plsc.load_gather(sbuf, [rowv, j2c]), neg)
            lm = jnp.max(jnp.maximum(jnp.maximum(g0, g1), g2))
            p0 = jnp.where(g0 == lm, j0, bigv)
            p1 = jnp.where(g1 == lm, j1, bigv)
            p2 = jnp.where(g2 == lm, j2c, bigv)
            lp = jnp.min(jnp.minimum(jnp.minimum(p0, p1), p2)
                         .astype(jnp.uint32)).astype(jnp.int32) + zeros16
            lmask = lanes == l
            vmax = jnp.where(lmask, lm, vmax)
            vidx = jnp.where(lmask, lp, vidx)
            return vmax, vidx, vreg

        _, _, vreg = lax.fori_loop(
            0, VOTES, extract, (vmax, vidx, jnp.zeros((16,), jnp.int32)))
        vbuf[pl.ds(pl.multiple_of(h * 16, 16), 16)] = vreg
        return 0

    lax.fori_loop(0, HW, head_body, 0)

    pltpu.sync_copy(vbuf, shared.at[pl.ds(pl.multiple_of(s_id * 96, 16), 96)])
    plsc.subcore_barrier()

    @pl.when(g == 0)
    def _():
        b = sm
        pltpu.sync_copy(
            shared.at[pl.ds(pl.multiple_of((s_id + 1) * 96, 16), 96)], vbuf2)

        def zero_chunk(ci, _):
            pbuf[pl.ds(ci * 16, 16)] = jnp.zeros((16,), jnp.float32)
            return 0

        lax.fori_loop(0, CONVPAD // 16, zero_chunk, 0)

        ones = jnp.ones((16,), jnp.float32)

        def scat(h, _):
            hb = pl.multiple_of(h * 16, 16)
            vv = vbuf[pl.ds(hb, 16)]
            plsc.addupdate_scatter(pbuf, [vv], ones, mask=lanes < VOTES)
            vv2 = vbuf2[pl.ds(hb, 16)]
            plsc.addupdate_scatter(pbuf, [vv2], ones, mask=lanes < VOTES)
            return 0

        lax.fori_loop(0, HW, scat, 0)

        def conv_chunk(ci, _):
            base = ci * 16
            left = pbuf[pl.ds(base, 16)]
            ctr = plsc.load_gather(pbuf, [base + 1 + lanes])
            right = plsc.load_gather(pbuf, [base + 2 + lanes])
            o = left + 2.0 * ctr + right
            obuf[pl.ds(base, 16)] = o
            kbuf[pl.ds(base, 16)] = (
                o.astype(jnp.int32) * 1024 + (1023 - (base + lanes)))
            return 0

        lax.fori_loop(0, P // 16, conv_chunk, 0)
        kbuf[pl.ds(P, 16)] = jnp.zeros((16,), jnp.int32)

        def kscan(ci, vk):
            return jnp.maximum(vk, kbuf[pl.ds(ci * 16, 16)])

        vk = lax.fori_loop(0, P // 16, kscan, jnp.zeros((16,), jnp.int32))

        def sel(t, carry):
            vk, ireg = carry
            gk = jnp.max(vk.astype(jnp.uint32)).astype(jnp.int32) + zeros16
            i = 1023 - (gk & 1023)
            ireg = jnp.where(lanes == t, i + 1, ireg)
            plsc.store_scatter(kbuf, [i], jnp.zeros((16,), jnp.int32),
                               mask=lanes == 0)
            l = i & 15
            j0 = l + 16 * lanes
            j1 = j0 + 256
            j2 = jnp.minimum(j0 + 512, KPAD - 1)
            g0 = plsc.load_gather(kbuf, [j0])
            g1 = plsc.load_gather(kbuf, [j1])
            g2 = plsc.load_gather(kbuf, [j2])
            lm = jnp.max(jnp.maximum(jnp.maximum(g0, g1), g2)
                         .astype(jnp.uint32)).astype(jnp.int32) + zeros16
            vk = jnp.where(lanes == l, lm, vk)
            return vk, ireg

        _, ireg = lax.fori_loop(0, TOPK, sel,
                                (vk, jnp.zeros((16,), jnp.int32)))
        ibuf[pl.ds(0, 16)] = ireg

        pltpu.sync_copy(ibuf.at[pl.ds(0, 16)],
                        idx_out.at[pl.ds(pl.multiple_of(b * 16, 16), 16)])
        pltpu.sync_copy(obuf.at[pl.ds(0, P)],
                        cnt_out.at[pl.ds(pl.multiple_of(b * P, 16), P)])


_mhv_sc = functools.partial(
    pl.kernel,
    out_type=(jax.ShapeDtypeStruct((B * 16,), jnp.int32),
              jax.ShapeDtypeStruct((B * P,), jnp.float32)),
    mesh=plsc.VectorSubcoreMesh(core_axis_name="c", subcore_axis_name="s"),
    compiler_params=pltpu.CompilerParams(needs_layout_passes=False),
    scratch_types=[
        pltpu.VMEM((HW * 8, T), jnp.float32),
        pltpu.VMEM((HW * 16,), jnp.int32),
        pltpu.VMEM((HW * 16,), jnp.int32),
        pltpu.VMEM((CONVPAD,), jnp.float32),
        pltpu.VMEM((KPAD,), jnp.float32),
        pltpu.VMEM((KPAD,), jnp.int32),
        pltpu.VMEM((16,), jnp.int32),
        pltpu.VMEM_SHARED((16 * 96,), jnp.int32),
        pltpu.SemaphoreType.DMA,
    ],
)(_sc_body)


def kernel(x, select_num):
    idx_flat, cnt_flat = _mhv_sc(x)
    idx = jnp.reshape(idx_flat, (B, 16))[:, :TOPK]
    cnt = jnp.reshape(cnt_flat, (B, P))
    col = jnp.minimum(jnp.arange(TOPK), jnp.asarray(select_num, jnp.int32) - 1)
    return jnp.take(idx, col, axis=1), cnt


# speedup vs baseline: 7.2719x; 7.2719x over previous
import functools

import jax
import jax.numpy as jnp
from jax import lax
from jax.experimental import pallas as pl
from jax.experimental.pallas import tpu as pltpu
from jax.experimental.pallas import tpu_sc as plsc

B = 16
H = 12
HW = 6
P = 576
VOTES = 8
TOPK = 12
CONVPAD = 608
KPAD = 592
BIG = 1 << 20


def _sc_body(x_hbm, idx_out, cnt_out, sbuf, vbuf, vbuf2, pbuf, obuf, kbuf,
             ibuf, shared):
    lanes = lax.iota(jnp.int32, 16)
    zeros16 = jnp.zeros((16,), jnp.int32)
    s_id = lax.axis_index("s")
    c_id = lax.axis_index("c")
    sm = c_id * 8 + (s_id >> 1)
    g = s_id & 1
    neg = jnp.full((16,), -2.0, jnp.float32)
    bigv = jnp.full((16,), BIG, jnp.int32)

    off = pl.multiple_of((sm * 2 + g) * (HW * P), 16)
    pltpu.sync_copy(x_hbm.at[pl.ds(off, HW * P)], sbuf)

    def extract_one(rbv, t, vmax, vidx, vreg):
        m = jnp.max(vmax)
        p = jnp.min(jnp.where(vmax == m, vidx, bigv)
                    .astype(jnp.uint32)).astype(jnp.int32) + zeros16
        vreg = jnp.where(lanes == t, p, vreg)
        plsc.store_scatter(sbuf, [rbv + p], neg, mask=lanes == 0)
        l = p & 15
        j0 = l + 16 * lanes
        j1 = j0 + 256
        j2 = jnp.minimum(j0 + 512, P - 1)
        g0 = plsc.load_gather(sbuf, [rbv + j0])
        g1 = plsc.load_gather(sbuf, [rbv + j1])
        g2 = jnp.where(lanes < 4, plsc.load_gather(sbuf, [rbv + j2]), neg)
        lm = jnp.max(jnp.maximum(jnp.maximum(g0, g1), g2))
        p0 = jnp.where(g0 == lm, j0, bigv)
        p1 = jnp.where(g1 == lm, j1, bigv)
        p2 = jnp.where(g2 == lm, j2, bigv)
        lp = jnp.min(jnp.minimum(jnp.minimum(p0, p1), p2)
                     .astype(jnp.uint32)).astype(jnp.int32) + zeros16
        lmask = lanes == l
        vmax = jnp.where(lmask, lm, vmax)
        vidx = jnp.where(lmask, lp, vidx)
        return vmax, vidx, vreg

    def pair_body(hp, _):
        rbA = pl.multiple_of(hp * (2 * P), 16)
        rbB = pl.multiple_of(hp * (2 * P) + P, 16)
        rbvA = jnp.full((16,), rbA, jnp.int32)
        rbvB = jnp.full((16,), rbB, jnp.int32)

        def scan_chunk(ci, carry):
            vmaxA, vidxA, vmaxB, vidxB = carry
            pos = ci * 16 + lanes
            vA = sbuf[pl.ds(rbA + ci * 16, 16)]
            vB = sbuf[pl.ds(rbB + ci * 16, 16)]
            gtA = vA > vmaxA
            gtB = vB > vmaxB
            return (jnp.where(gtA, vA, vmaxA), jnp.where(gtA, pos, vidxA),
                    jnp.where(gtB, vB, vmaxB), jnp.where(gtB, pos, vidxB))

        neg3 = jnp.full((16,), -3.0, jnp.float32)
        vmaxA, vidxA, vmaxB, vidxB = lax.fori_loop(
            0, P // 16, scan_chunk, (neg3, zeros16, neg3, zeros16))

        def extract(t, carry):
            vmaxA, vidxA, vregA, vmaxB, vidxB, vregB = carry
            vmaxA, vidxA, vregA = extract_one(rbvA, t, vmaxA, vidxA, vregA)
            vmaxB, vidxB, vregB = extract_one(rbvB, t, vmaxB, vidxB, vregB)
            return vmaxA, vidxA, vregA, vmaxB, vidxB, vregB

        _, _, vregA, _, _, vregB = lax.fori_loop(
            0, VOTES, extract,
            (vmaxA, vidxA, zeros16, vmaxB, vidxB, zeros16))
        vb = pl.multiple_of(hp * 32, 16)
        vbuf[pl.ds(vb, 16)] = vregA
        vbuf[pl.ds(vb + 16, 16)] = vregB
        return 0

    lax.fori_loop(0, HW // 2, pair_body, 0)

    pltpu.sync_copy(vbuf, shared.at[pl.ds(pl.multiple_of(s_id * 96, 16), 96)])
    plsc.subcore_barrier()

    @pl.when(g == 0)
    def _():
        b = sm
        pltpu.sync_copy(
            shared.at[pl.ds(pl.multiple_of((s_id + 1) * 96, 16), 96)], vbuf2)

        def zero_chunk(ci, _):
            pbuf[pl.ds(ci * 16, 16)] = jnp.zeros((16,), jnp.float32)
            return 0

        lax.fori_loop(0, CONVPAD // 16, zero_chunk, 0)

        ones = jnp.ones((16,), jnp.float32)

        def scat(h, _):
            hb = pl.multiple_of(h * 16, 16)
            vv = vbuf[pl.ds(hb, 16)]
            plsc.addupdate_scatter(pbuf, [vv + 1], ones, mask=lanes < VOTES)
            vv2 = vbuf2[pl.ds(hb, 16)]
            plsc.addupdate_scatter(pbuf, [vv2 + 1], ones, mask=lanes < VOTES)
            return 0

        lax.fori_loop(0, HW, scat, 0)

        def conv_chunk(ci, _):
            base = ci * 16
            left = pbuf[pl.ds(base, 16)]
            ctr = plsc.load_gather(pbuf, [base + 1 + lanes])
            right = plsc.load_gather(pbuf, [base + 2 + lanes])
            o = left + 2.0 * ctr + right
            obuf[pl.ds(base, 16)] = o
            kbuf[pl.ds(base, 16)] = (
                o.astype(jnp.int32) * 1024 + (1023 - (base + lanes)))
            return 0

        lax.fori_loop(0, P // 16, conv_chunk, 0)
        kbuf[pl.ds(P, 16)] = jnp.zeros((16,), jnp.int32)

        def kscan(ci, vk):
            return jnp.maximum(vk, kbuf[pl.ds(ci * 16, 16)])

        vk = lax.fori_loop(0, P // 16, kscan, jnp.zeros((16,), jnp.int32))

        def sel(t, carry):
            vk, ireg = carry
            gk = jnp.max(vk.astype(jnp.uint32)).astype(jnp.int32) + zeros16
            i = 1023 - (gk & 1023)
            ireg = jnp.where(lanes == t, i + 1, ireg)
            plsc.store_scatter(kbuf, [i], jnp.zeros((16,), jnp.int32),
                               mask=lanes == 0)
            l = i & 15
            j0 = l + 16 * lanes
            j1 = j0 + 256
            j2 = jnp.minimum(j0 + 512, KPAD - 1)
            g0 = plsc.load_gather(kbuf, [j0])
            g1 = plsc.load_gather(kbuf, [j1])
            g2 = plsc.load_gather(kbuf, [j2])
            lm = jnp.max(jnp.maximum(jnp.maximum(g0, g1), g2)
                         .astype(jnp.uint32)).astype(jnp.int32) + zeros16
            vk = jnp.where(lanes == l, lm, vk)
            return vk, ireg

        _, ireg = lax.fori_loop(0, TOPK, sel,
                                (vk, jnp.zeros((16,), jnp.int32)))
        ibuf[pl.ds(0, 16)] = ireg

        pltpu.sync_copy(ibuf.at[pl.ds(0, 16)],
                        idx_out.at[pl.ds(pl.multiple_of(b * 16, 16), 16)])
        pltpu.sync_copy(obuf.at[pl.ds(0, P)],
                        cnt_out.at[pl.ds(pl.multiple_of(b * P, 16), P)])


_mhv_sc = functools.partial(
    pl.kernel,
    out_type=(jax.ShapeDtypeStruct((B * 16,), jnp.int32),
              jax.ShapeDtypeStruct((B * P,), jnp.float32)),
    mesh=plsc.VectorSubcoreMesh(core_axis_name="c", subcore_axis_name="s"),
    compiler_params=pltpu.CompilerParams(needs_layout_passes=False),
    scratch_types=[
        pltpu.VMEM((HW * P,), jnp.float32),
        pltpu.VMEM((HW * 16,), jnp.int32),
        pltpu.VMEM((HW * 16,), jnp.int32),
        pltpu.VMEM((CONVPAD,), jnp.float32),
        pltpu.VMEM((KPAD,), jnp.float32),
        pltpu.VMEM((KPAD,), jnp.int32),
        pltpu.VMEM((16,), jnp.int32),
        pltpu.VMEM_SHARED((16 * 96,), jnp.int32),
    ],
)(_sc_body)


def kernel(x, select_num):
    score = x[:, :, 0, 1:]
    idx_flat, cnt_flat = _mhv_sc(jnp.reshape(score, (-1,)))
    idx = jnp.reshape(idx_flat, (B, 16))[:, :TOPK]
    cnt = jnp.reshape(cnt_flat, (B, P))
    col = jnp.minimum(jnp.arange(TOPK), jnp.asarray(select_num, jnp.int32) - 1)
    return jnp.take(idx, col, axis=1), cnt
